# baseline (speedup 1.0000x reference)
import functools

import jax
import jax.numpy as jnp
from jax import lax
from jax.experimental import pallas as pl
from jax.experimental.pallas import tpu as pltpu
from jax.experimental.pallas import tpu_sc as plsc

f32 = jnp.float32
i32 = jnp.int32

N = 10000
E = 160000
C = 32
NC = 1000
EC = 4000

E_PAD = 163840
A_PAD = 10016
EC_PAD = 4096
AC_PAD = 1024
EG_PAD = 131072
AG_PAD = 32016
OP_PAD = 12288
IDS_PAD = 1024

_MESH = plsc.VectorSubcoreMesh(core_axis_name="c", subcore_axis_name="s")
_SC_PARAMS = pltpu.CompilerParams(use_tc_tiling_on_sc=False)


def _leak(x):
    return jnp.where(x >= 0, x, 0.01 * x)


_Q = 4
_D = 2


def _edge_loop(table, src_v, dst_v, rows, acc, gsems, ssems, nch,
               ones_v=None, accc=None, csem=None, q=_Q, d=_D):
    if nch < q:
        def chunk(j, carry):
            pltpu.async_copy(table.at[src_v.at[j]], rows.at[0],
                             gsems[0]).wait()
            pltpu.sync_copy(rows.at[0], acc.at[dst_v.at[j]], add=True)
            if ones_v is not None:
                pltpu.sync_copy(ones_v, accc.at[dst_v.at[j]], add=True)
            return carry

        lax.fori_loop(0, nch, chunk, 0)
        return

    def visit(j, b):
        @pl.when(j >= q)
        def _():
            pltpu.make_async_copy(rows.at[b], acc.at[dst_v.at[0]],
                                  ssems[b]).wait()

        pltpu.async_copy(table.at[src_v.at[j]], rows.at[b], gsems[b])

        @pl.when(j >= d)
        def _():
            jd = j - d
            qd = (b + q - d) % q
            pltpu.make_async_copy(table.at[src_v.at[0]], rows.at[qd],
                                  gsems[qd]).wait()
            pltpu.async_copy(rows.at[qd], acc.at[dst_v.at[jd]],
                             ssems[qd], add=True)
            if ones_v is not None:
                pltpu.async_copy(ones_v, accc.at[dst_v.at[jd]], csem, add=True)

    def group(g, carry):
        for b in range(q):
            visit(g * q + b, b)
        return carry

    lax.fori_loop(0, nch // q, group, 0)
    for jd in range(nch - d, nch):
        qd = jd % q
        pltpu.make_async_copy(table.at[src_v.at[0]], rows.at[qd],
                              gsems[qd]).wait()
        pltpu.async_copy(rows.at[qd], acc.at[dst_v.at[jd]],
                         ssems[qd], add=True)
        if ones_v is not None:
            pltpu.async_copy(ones_v, accc.at[dst_v.at[jd]], csem, add=True)
    for b in range(q):
        pltpu.make_async_copy(rows.at[b], acc.at[dst_v.at[0]],
                              ssems[b]).wait()
    if ones_v is not None:
        def drain(j, carry):
            pltpu.make_async_copy(ones_v, accc.at[dst_v.at[0]],
                                  csem).wait()
            return carry
        lax.fori_loop(0, nch, drain, 0)

@functools.lru_cache(maxsize=None)
def _sc_segsum(nch, a_pad, with_counts, a16_pad=0, stage_rows=0, q=_Q, d=_D):
    zrows = a_pad // 16
    out_type = [jax.ShapeDtypeStruct((2, a_pad, 64), f32)]
    scratch = [
        pltpu.VMEM((nch, 128), i32),
        pltpu.VMEM((nch, 128), i32),
        pltpu.VMEM((q, 128, 64), f32),
    ] + [pltpu.SemaphoreType.DMA] * (2 * q) + [
        pltpu.VMEM_SHARED((a_pad, 64), f32),
    ]
    if stage_rows:
        scratch.append(pltpu.VMEM_SHARED((stage_rows, 64), f32))
    if with_counts:
        z16 = a16_pad // 16
        out_type.append(jax.ShapeDtypeStruct((2, a16_pad, 16), f32))
        scratch += [
            pltpu.VMEM((128, 16), f32),
            pltpu.SemaphoreType.DMA,
            pltpu.VMEM_SHARED((a16_pad, 16), f32),
        ]

    def body(*refs):
        if with_counts:
            (table, src2d, dst2d, zeros64, zeros16, ones16,
             out, outc, src_v, dst_v, rows, *rest) = refs
            gsems, ssems = rest[:q], rest[q:2 * q]
            if stage_rows:
                acc, table_sh, ones_v, csem, accc = rest[2 * q:]
            else:
                acc, ones_v, csem, accc = rest[2 * q:]
                table_sh = None
        else:
            (table, src2d, dst2d, zeros64,
             out, src_v, dst_v, rows, *rest) = refs
            gsems, ssems = rest[:q], rest[q:2 * q]
            acc = rest[2 * q]
            table_sh = rest[2 * q + 1] if stage_rows else None
            ones_v = csem = accc = None
        cid = lax.axis_index("c")
        sid = lax.axis_index("s")
        wid = cid * 16 + sid
        if stage_rows:
            srows = stage_rows // 16
            pltpu.sync_copy(table.at[pl.ds(sid * srows, srows)],
                            table_sh.at[pl.ds(sid * srows, srows)])
            table = table_sh
        pltpu.sync_copy(zeros64.at[pl.ds(0, zrows)],
                        acc.at[pl.ds(sid * zrows, zrows)])
        if with_counts:
            z16 = a16_pad // 16
            pltpu.sync_copy(zeros16.at[pl.ds(0, z16)],
                            accc.at[pl.ds(sid * z16, z16)])
            pltpu.sync_copy(ones16, ones_v)
        pltpu.sync_copy(src2d.at[pl.ds(wid * nch, nch)], src_v)
        pltpu.sync_copy(dst2d.at[pl.ds(wid * nch, nch)], dst_v)
        plsc.subcore_barrier()
        _edge_loop(table, src_v, dst_v, rows, acc, gsems, ssems, nch,
                   ones_v=ones_v, accc=accc, csem=csem, q=q, d=d)
        plsc.subcore_barrier()
        pltpu.sync_copy(acc.at[pl.ds(sid * zrows, zrows)],
                        out.at[cid].at[pl.ds(sid * zrows, zrows)])
        if with_counts:
            z16 = a16_pad // 16
            pltpu.sync_copy(accc.at[pl.ds(sid * z16, z16)],
                            outc.at[cid].at[pl.ds(sid * z16, z16)])

    k = pl.kernel(body, out_type=out_type, mesh=_MESH, scratch_types=scratch,
                  compiler_params=_SC_PARAMS)
    if with_counts:
        return k
    return lambda *a: k(*a)[0]


@functools.lru_cache(maxsize=None)
def _sc_seg_percfg(nch):
    a_pad = 16 * NC + 16
    zrows = a_pad // 16

    def body(table, src2d, dst2d, zeros, out, src_v, dst_v, rows, *rest):
        gsems, ssems = rest[:_Q], rest[_Q:2 * _Q]
        acc = rest[2 * _Q]
        cid = lax.axis_index("c")
        sid = lax.axis_index("s")
        wid = cid * 16 + sid
        pltpu.sync_copy(zeros.at[pl.ds(0, zrows)],
                        acc.at[pl.ds(sid * zrows, zrows)])
        pltpu.sync_copy(src2d.at[pl.ds(wid * nch, nch)], src_v)
        pltpu.sync_copy(dst2d.at[pl.ds(sid * nch, nch)], dst_v)
        plsc.subcore_barrier()
        _edge_loop(table, src_v, dst_v, rows, acc, gsems, ssems, nch)
        plsc.subcore_barrier()
        pltpu.sync_copy(acc.at[pl.ds(sid * NC, NC)],
                        out.at[pl.ds(wid * NC, NC)])

    return pl.kernel(
        body,
        out_type=jax.ShapeDtypeStruct((C * NC, 64), f32),
        mesh=_MESH,
        scratch_types=[
            pltpu.VMEM((nch, 128), i32),
            pltpu.VMEM((nch, 128), i32),
            pltpu.VMEM((_Q, 128, 64), f32),
        ] + [pltpu.SemaphoreType.DMA] * (2 * _Q) + [
            pltpu.VMEM_SHARED((a_pad, 64), f32),
        ],
        compiler_params=_SC_PARAMS,
    )


@functools.lru_cache(maxsize=None)
def _sc_seg_final():
    nch = E_PAD // 4096
    zrows = A_PAD // 16
    srows = N // 16

    def body(table, src2d, dst2d, zeros64, ids2d, cnt0h, cnt1h,
             pg, xg, c0g, c1g, src_v, dst_v, rows, idx_v, cbuf, *rest):
        gsems, ssems = rest[:_Q], rest[_Q:2 * _Q]
        acc, table_sh = rest[2 * _Q], rest[2 * _Q + 1]
        cid = lax.axis_index("c")
        sid = lax.axis_index("s")
        wid = cid * 16 + sid
        pltpu.sync_copy(table.at[pl.ds(sid * srows, srows)],
                        table_sh.at[pl.ds(sid * srows, srows)])
        pltpu.sync_copy(zeros64.at[pl.ds(0, zrows)],
                        acc.at[pl.ds(sid * zrows, zrows)])
        pltpu.sync_copy(src2d.at[pl.ds(wid * nch, nch)], src_v)
        pltpu.sync_copy(dst2d.at[pl.ds(wid * nch, nch)], dst_v)
        plsc.subcore_barrier()
        _edge_loop(table_sh, src_v, dst_v, rows, acc, gsems, ssems, nch)
        plsc.subcore_barrier()

        nids = IDS_PAD // 128

        @pl.when(sid < nids)
        def _():
            pltpu.sync_copy(ids2d.at[pl.ds(sid, 1)], idx_v)
            pltpu.async_copy(acc.at[idx_v.at[0]], rows.at[0],
                             gsems[0]).wait()
            pltpu.sync_copy(rows.at[0],
                            pg.at[cid].at[pl.ds(sid * 128, 128)])

            @pl.when(cid == 0)
            def _():
                pltpu.async_copy(table_sh.at[idx_v.at[0]], rows.at[1],
                                 gsems[1]).wait()
                pltpu.sync_copy(rows.at[1], xg.at[pl.ds(sid * 128, 128)])

        @pl.when((sid >= nids) & (sid < 2 * nids))
        def _():
            s8 = sid - nids
            pltpu.sync_copy(ids2d.at[pl.ds(s8, 1)], idx_v)

            @pl.when(cid == 0)
            def _():
                pltpu.async_copy(cnt0h.at[idx_v.at[0]], cbuf,
                                 gsems[2]).wait()
                pltpu.sync_copy(cbuf, c0g.at[pl.ds(s8 * 128, 128)])

            @pl.when(cid == 1)
            def _():
                pltpu.async_copy(cnt1h.at[idx_v.at[0]], cbuf,
                                 gsems[2]).wait()
                pltpu.sync_copy(cbuf, c1g.at[pl.ds(s8 * 128, 128)])

    return pl.kernel(
        body,
        out_type=[
            jax.ShapeDtypeStruct((2, IDS_PAD, 64), f32),
            jax.ShapeDtypeStruct((IDS_PAD, 64), f32),
            jax.ShapeDtypeStruct((IDS_PAD, 16), f32),
            jax.ShapeDtypeStruct((IDS_PAD, 16), f32),
        ],
        mesh=_MESH,
        scratch_types=[
            pltpu.VMEM((nch, 128), i32),
            pltpu.VMEM((nch, 128), i32),
            pltpu.VMEM((_Q, 128, 64), f32),
            pltpu.VMEM((1, 128), i32),
            pltpu.VMEM((128, 16), f32),
        ] + [pltpu.SemaphoreType.DMA] * (2 * _Q) + [
            pltpu.VMEM_SHARED((A_PAD, 64), f32),
            pltpu.VMEM_SHARED((N, 64), f32),
        ],
        compiler_params=_SC_PARAMS,
    )


@functools.lru_cache(maxsize=None)
def _sc_gather(nrows, d):
    rpt = max(nrows // 32, 1)

    def body(table, idx2d, out, idx_v, rows_v, sem):
        cid = lax.axis_index("c")
        sid = lax.axis_index("s")
        wid = cid * 16 + sid
        for r in range(rpt):
            j = wid * rpt + r

            @pl.when(j < nrows)
            def _():
                pltpu.sync_copy(idx2d.at[pl.ds(j, 1)], idx_v)
                pltpu.async_copy(table.at[idx_v.at[0]], rows_v, sem).wait()
                pltpu.sync_copy(rows_v, out.at[pl.ds(j * 128, 128)])

    return pl.kernel(
        body,
        out_type=jax.ShapeDtypeStruct((nrows * 128, d), f32),
        mesh=_MESH,
        compiler_params=_SC_PARAMS,
        scratch_types=[
            pltpu.VMEM((1, 128), i32),
            pltpu.VMEM((128, d), f32),
            pltpu.SemaphoreType.DMA,
        ],
    )


def _dot(a, b):
    return jnp.dot(a, b, preferred_element_type=f32)


def _tc_proj2(nf, eg, wla, wlb, wra, wrb):
    def body(nf_r, eg_r, wla_r, wlb_r, wra_r, wrb_r, l_r, r_r):
        a = nf_r[...]
        e = eg_r[...]
        l_r[...] = _dot(a, wla_r[...]) + _dot(e, wlb_r[...])
        r_r[...] = _dot(a, wra_r[...]) + _dot(e, wrb_r[...])

    n = nf.shape[0]
    return pl.pallas_call(
        body,
        out_shape=[jax.ShapeDtypeStruct((n, 64), f32)] * 2,
    )(nf, eg, wla, wlb, wra, wrb)


def _tc_proj(x, wl, wr):
    def body(x_r, wl_r, wr_r, l_r, r_r):
        v = x_r[...]
        l_r[...] = _dot(v, wl_r[...])
        r_r[...] = _dot(v, wr_r[...])

    n = x.shape[0]
    return pl.pallas_call(
        body,
        out_shape=[jax.ShapeDtypeStruct((n, 64), f32)] * 2,
    )(x, wl, wr)


def _tc_combine(s0, s1, c0, c1, r, bl, wl=None, wr=None):
    project = wl is not None

    def body(*refs):
        if project:
            s0_r, s1_r, c0_r, c1_r, r_r, bl_r, wl_r, wr_r, *outs = refs
        else:
            s0_r, s1_r, c0_r, c1_r, r_r, bl_r, *outs = refs
        cnt = jnp.maximum(c0_r[...][:, :1] + c1_r[...][:, :1], 1.0)
        x = (s0_r[...] + s1_r[...]) / cnt + bl_r[...] + r_r[...]
        x = _leak(x)
        if project:
            outs[0][...] = _dot(x, wl_r[...])
            outs[1][...] = _dot(x, wr_r[...])
        else:
            outs[0][...] = x

    n = s0.shape[0]
    if project:
        return pl.pallas_call(
            body, out_shape=[jax.ShapeDtypeStruct((n, 64), f32)] * 2,
        )(s0, s1, c0, c1, r, bl, wl, wr)
    return pl.pallas_call(
        body, out_shape=jax.ShapeDtypeStruct((n, 64), f32),
    )(s0, s1, c0, c1, r, bl)


def _tc_xcat(x3, p0, p1, c0, c1):
    def body(x_r, p0_r, p1_r, c0_r, c1_r, o_r):
        cnt = jnp.maximum(c0_r[...][:, :1] + c1_r[...][:, :1], 1.0)
        o_r[:, :64] = x_r[...]
        o_r[:, 64:] = (p0_r[...] + p1_r[...]) / cnt

    return pl.pallas_call(
        body, out_shape=jax.ShapeDtypeStruct((N, 128), f32),
    )(x3, p0, p1, c0, c1)


def _tc_cfgstart(p0, p1, c0, c1, wl, wr):
    def body(p0_r, p1_r, c0_r, c1_r, wl_r, wr_r, l_r, r_r):
        cnt = jnp.maximum(c0_r[...][:, :1] + c1_r[...][:, :1], 1.0)
        nb = (p0_r[...] + p1_r[...]) / cnt
        l_r[...] = _dot(nb, wl_r[...])
        r_r[...] = _dot(nb, wr_r[...])

    n = p0.shape[0]
    return pl.pallas_call(
        body, out_shape=[jax.ShapeDtypeStruct((n, 64), f32)] * 2,
    )(p0, p1, c0, c1, wl, wr)


def _tc_ncfpre(ncfeat, prjw, prjb, wl_c, wr_c):
    def body(nc_r, pw_r, pb_r, wlc_r, wrc_r, bl_r, br_r, n2_r):
        v = nc_r[0]
        ncf = _leak(_dot(v, pw_r[...]) + pb_r[...])
        bl_r[0] = _dot(ncf, wlc_r[...])
        br_r[0] = _dot(ncf, wrc_r[...])
        n2_r[0] = jnp.sum(ncf * ncf, axis=1, keepdims=True)

    full = lambda shp: pl.BlockSpec(shp, lambda c: tuple(0 for _ in shp))
    return pl.pallas_call(
        body,
        grid=(C,),
        in_specs=[
            pl.BlockSpec((1, NC, 18), lambda c: (c, 0, 0)),
            full((18, 64)), full((1, 64)),
            full((64, 64)), full((64, 64)),
        ],
        out_specs=[pl.BlockSpec((1, NC, 64), lambda c: (c, 0, 0))] * 2
        + [pl.BlockSpec((1, NC, 1), lambda c: (c, 0, 0))],
        out_shape=[jax.ShapeDtypeStruct((C, NC, 64), f32)] * 2
        + [jax.ShapeDtypeStruct((C, NC, 1), f32)],
    )(ncfeat, prjw, prjb, wl_c, wr_c)


def _tc_cfgprep(t0, t1, cc0, cc1, t_r, cbl3, xc, ncf_bl, ncf_br, ncf_n2,
                wl_a, wl_b, wr_a, wr_b):
    def body(t0_r, t1_r, cc0_r, cc1_r, tr_r, cbl_r, xc_r,
             nbl_r, nbr_r, nn2_r, wla_r, wlb_r, wra_r, wrb_r, yl_r, yr_r):
        cnt = jnp.maximum(cc0_r[...][:, :1] + cc1_r[...][:, :1], 1.0)
        nbv = _leak((t0_r[...] + t1_r[...]) / cnt + cbl_r[...] + tr_r[...])
        xcv = xc_r[...]
        n2 = (jnp.sum(nbv * nbv, axis=1, keepdims=True)
              + jnp.sum(xcv * xcv, axis=1, keepdims=True)
              + nn2_r[0])
        inv = 1.0 / jnp.maximum(jnp.sqrt(n2), 1e-12)
        al = _dot(nbv, wla_r[...]) + _dot(xcv, wlb_r[...])
        ar = _dot(nbv, wra_r[...]) + _dot(xcv, wrb_r[...])
        yl_r[0] = (al + nbl_r[0]) * inv
        yr_r[0] = (ar + nbr_r[0]) * inv

    full = lambda shp: pl.BlockSpec(shp, lambda c: tuple(0 for _ in shp))
    cfg_blk = pl.BlockSpec((1, NC, 64), lambda c: (c, 0, 0))
    return pl.pallas_call(
        body,
        grid=(C,),
        in_specs=[
            full((NC, 64)), full((NC, 64)), full((NC, 16)), full((NC, 16)),
            full((NC, 64)), full((1, 64)), full((NC, 64)),
            cfg_blk, cfg_blk, pl.BlockSpec((1, NC, 1), lambda c: (c, 0, 0)),
            full((64, 64)), full((64, 64)), full((64, 64)), full((64, 64)),
        ],
        out_specs=[cfg_blk] * 2,
        out_shape=[jax.ShapeDtypeStruct((C, NC, 64), f32)] * 2,
    )(t0, t1, cc0, cc1, t_r, cbl3, xc, ncf_bl, ncf_br, ncf_n2,
      wl_a, wl_b, wr_a, wr_b)


def _tc_cfg_combine(s, c0, c1, yr, bl, wl, wr):
    def body(s_r, c0_r, c1_r, yr_r, bl_r, wl_r, wr_r, l_r, r_r):
        cnt = jnp.maximum(c0_r[...][:, :1] + c1_r[...][:, :1], 1.0)
        x = s_r[0] / cnt + bl_r[...] + yr_r[0]
        x = _leak(x)
        l_r[0] = _dot(x, wl_r[...])
        r_r[0] = _dot(x, wr_r[...])

    full = lambda shp: pl.BlockSpec(shp, lambda c: tuple(0 for _ in shp))
    cfg_blk = pl.BlockSpec((1, NC, 64), lambda c: (c, 0, 0))
    return pl.pallas_call(
        body,
        grid=(C,),
        in_specs=[cfg_blk, full((NC, 16)), full((NC, 16)), cfg_blk,
                  full((1, 64)), full((64, 64)), full((64, 64))],
        out_specs=[cfg_blk] * 2,
        out_shape=[jax.ShapeDtypeStruct((C, NC, 64), f32)] * 2,
    )(s, c0, c1, yr, bl, wl, wr)


def _tc_final(s, c0, c1, yr, bl, d1, d2, d3):
    def pool_body(s_r, c0_r, c1_r, yr_r, bl_r, o_r):
        cnt = jnp.maximum(c0_r[...][:, :1] + c1_r[...][:, :1], 1.0)
        h = s_r[0] / cnt + bl_r[...] + yr_r[0]
        h = _leak(h)
        o_r[0] = jnp.mean(h, axis=0, keepdims=True)

    full = lambda shp: pl.BlockSpec(shp, lambda c: tuple(0 for _ in shp))
    cfg_blk = pl.BlockSpec((1, NC, 64), lambda c: (c, 0, 0))
    pooled = pl.pallas_call(
        pool_body,
        grid=(C,),
        in_specs=[cfg_blk, full((NC, 16)), full((NC, 16)), cfg_blk,
                  full((1, 64))],
        out_specs=pl.BlockSpec((1, 1, 64), lambda c: (c, 0, 0)),
        out_shape=jax.ShapeDtypeStruct((C, 1, 64), f32),
    )(s, c0, c1, yr, bl).reshape(C, 64)

    def head_body(p_r, d1_r, d2_r, d3_r, o_r):
        y = _leak(_dot(p_r[...], d1_r[...]))
        y = _leak(_dot(y, d2_r[...]))
        o_r[...] = _dot(y, d3_r[...])

    return pl.pallas_call(
        head_body, out_shape=jax.ShapeDtypeStruct((C, 1), f32),
    )(pooled, d1, d2, d3)


def kernel(node_feat, node_opcode, edge_index, node_config_feat,
           node_config_ids, config_edge_index, params):
    p = params
    src = edge_index[0].astype(i32)
    dst = edge_index[1].astype(i32)
    csrc = config_edge_index[0].astype(i32)
    cdst = config_edge_index[1].astype(i32)
    opc = node_opcode.astype(i32)
    ids = node_config_ids.astype(i32)

    srcp = jnp.concatenate([src, jnp.zeros((E_PAD - E,), i32)]).reshape(-1, 128)
    dstp = jnp.concatenate([dst, jnp.full((E_PAD - E,), N, i32)]).reshape(-1, 128)
    opcp = jnp.concatenate([opc, jnp.zeros((OP_PAD - N,), i32)]).reshape(-1, 128)
    idsp = jnp.concatenate([ids, jnp.zeros((IDS_PAD - NC,), i32)]).reshape(-1, 128)
    csrcp = jnp.concatenate([csrc, jnp.zeros((EC_PAD - EC,), i32)]).reshape(-1, 128)
    cdstp = jnp.concatenate([cdst, jnp.full((EC_PAD - EC,), NC, i32)]).reshape(-1, 128)
    coff = (jnp.arange(C, dtype=i32) * NC)[:, None]
    gsrc = (csrcp.reshape(-1)[None, :] + coff).reshape(-1, 128)
    pad_mask = jnp.arange(EC_PAD) >= EC
    cdst0 = jnp.concatenate([cdst, jnp.zeros((EC_PAD - EC,), i32)])
    soff = (jnp.arange(16, dtype=i32) * NC)[:, None]
    dstcfg = jnp.where(pad_mask[None, :], 16 * NC,
                       cdst0[None, :] + soff).reshape(-1, 128)

    zeros64 = jnp.zeros((AG_PAD // 16, 64), f32)
    zeros16 = jnp.zeros((A_PAD // 16, 16), f32)
    ones16 = jnp.ones((128, 16), f32)

    ng, cg, gg = p["node_gnn"], p["cfg_nbr_gnn"], p["cfg_gnn"]
    wl1, wr1 = ng[0]["Wl"], ng[0]["Wr"]
    bl = [lyr["bl"].reshape(1, 64) for lyr in ng]
    cbl = [lyr["bl"].reshape(1, 64) for lyr in cg]
    gbl = [lyr["bl"].reshape(1, 64) for lyr in gg]
    gwl1, gwr1 = gg[0]["Wl"], gg[0]["Wr"]

    ncf_bl, ncf_br, ncf_n2 = _tc_ncfpre(
        node_config_feat, p["prj_W"], p["prj_b"].reshape(1, 64),
        gwl1[128:], gwr1[128:])

    embg = _sc_gather(OP_PAD // 128, 32)(p["embedding"], opcp)[:N]
    l1, r1 = _tc_proj2(node_feat, embg, wl1[:140], wl1[140:], wr1[:140], wr1[140:])
    s1, cnt = _sc_segsum(E_PAD // 4096, A_PAD, True, A_PAD, N, 2, 1)(
        l1, srcp, dstp, zeros64, zeros16, ones16)
    c0, c1 = cnt[0, :N], cnt[1, :N]
    l2, r2 = _tc_combine(s1[0, :N], s1[1, :N], c0, c1, r1, bl[0],
                         ng[1]["Wl"], ng[1]["Wr"])
    s2 = _sc_segsum(E_PAD // 4096, A_PAD, False, 0, N)(l2, srcp, dstp, zeros64)
    l3, r3 = _tc_combine(s2[0, :N], s2[1, :N], c0, c1, r2, bl[1],
                         ng[2]["Wl"], ng[2]["Wr"])
    s3 = _sc_segsum(E_PAD // 4096, A_PAD, False, 0, N)(l3, srcp, dstp, zeros64)
    x3 = _tc_combine(s3[0, :N], s3[1, :N], c0, c1, r3, bl[2])
    pg, xg, c0gf, c1gf = _sc_seg_final()(
        x3, srcp, dstp, zeros64, idsp, cnt[0], cnt[1])
    xc = xg[:NC]

    t_l, t_r = _tc_cfgstart(pg[0, :NC], pg[1, :NC], c0gf[:NC], c1gf[:NC],
                            cg[0]["Wl"], cg[0]["Wr"])
    t1, ccnt = _sc_segsum(EC_PAD // 4096, AC_PAD, True, AC_PAD)(
        t_l, csrcp, cdstp, zeros64, zeros16, ones16)
    cc0, cc1 = ccnt[0, :NC], ccnt[1, :NC]
    t_l, t_r = _tc_combine(t1[0, :NC], t1[1, :NC], cc0, cc1, t_r, cbl[0],
                           cg[1]["Wl"], cg[1]["Wr"])
    t2 = _sc_segsum(EC_PAD // 4096, AC_PAD, False)(t_l, csrcp, cdstp, zeros64)
    t_l, t_r = _tc_combine(t2[0, :NC], t2[1, :NC], cc0, cc1, t_r, cbl[1],
                           cg[2]["Wl"], cg[2]["Wr"])
    t3 = _sc_segsum(EC_PAD // 4096, AC_PAD, False)(t_l, csrcp, cdstp, zeros64)

    yl, yr = _tc_cfgprep(t3[0, :NC], t3[1, :NC], cc0, cc1, t_r, cbl[2], xc,
                         ncf_bl, ncf_br, ncf_n2,
                         gwl1[:64], gwl1[64:128], gwr1[:64], gwr1[64:128])
    g1 = _sc_seg_percfg(EG_PAD // 4096)(
        yl.reshape(C * NC, 64), gsrc, dstcfg, zeros64).reshape(C, NC, 64)
    yl, yr = _tc_cfg_combine(g1, cc0, cc1, yr, gbl[0],
                             gg[1]["Wl"], gg[1]["Wr"])
    g2 = _sc_seg_percfg(EG_PAD // 4096)(
        yl.reshape(C * NC, 64), gsrc, dstcfg, zeros64).reshape(C, NC, 64)
    yl, yr = _tc_cfg_combine(g2, cc0, cc1, yr, gbl[1],
                             gg[2]["Wl"], gg[2]["Wr"])
    g3 = _sc_seg_percfg(EG_PAD // 4096)(
        yl.reshape(C * NC, 64), gsrc, dstcfg, zeros64).reshape(C, NC, 64)
    y = _tc_final(g3, cc0, cc1, yr, gbl[2],
                  p["d1"], p["d2"], p["d3"])
    return y.reshape(-1)


# speedup vs baseline: 1.0454x; 1.0454x over previous
import functools

import jax
import jax.numpy as jnp
from jax import lax
from jax.experimental import pallas as pl
from jax.experimental.pallas import tpu as pltpu
from jax.experimental.pallas import tpu_sc as plsc

f32 = jnp.float32
i32 = jnp.int32

N = 10000
E = 160000
C = 32
NC = 1000
EC = 4000

E_PAD = 163840
A_PAD = 10016
EC_PAD = 4096
AC_PAD = 1024
EG_PAD = 131072
AG_PAD = 32016
OP_PAD = 12288
IDS_PAD = 1024

_MESH = plsc.VectorSubcoreMesh(core_axis_name="c", subcore_axis_name="s")
_SC_PARAMS = pltpu.CompilerParams(use_tc_tiling_on_sc=False)


def _leak(x):
    return jnp.where(x >= 0, x, 0.01 * x)


_Q = 4
_D = 2


def _edge_loop(table, src_v, dst_v, rows, acc, gsems, ssems, nch,
               ones_v=None, accc=None, csem=None, q=_Q, d=_D):
    if nch < q:
        def chunk(j, carry):
            pltpu.async_copy(table.at[src_v.at[j]], rows.at[0],
                             gsems[0]).wait()
            pltpu.sync_copy(rows.at[0], acc.at[dst_v.at[j]], add=True)
            if ones_v is not None:
                pltpu.sync_copy(ones_v, accc.at[dst_v.at[j]], add=True)
            return carry

        lax.fori_loop(0, nch, chunk, 0)
        return

    def visit(j, b):
        @pl.when(j >= q)
        def _():
            pltpu.make_async_copy(rows.at[b], acc.at[dst_v.at[0]],
                                  ssems[b]).wait()

        pltpu.async_copy(table.at[src_v.at[j]], rows.at[b], gsems[b])

        @pl.when(j >= d)
        def _():
            jd = j - d
            qd = (b + q - d) % q
            pltpu.make_async_copy(table.at[src_v.at[0]], rows.at[qd],
                                  gsems[qd]).wait()
            pltpu.async_copy(rows.at[qd], acc.at[dst_v.at[jd]],
                             ssems[qd], add=True)
            if ones_v is not None:
                pltpu.async_copy(ones_v, accc.at[dst_v.at[jd]], csem, add=True)

    def group(g, carry):
        for b in range(q):
            visit(g * q + b, b)
        return carry

    lax.fori_loop(0, nch // q, group, 0)
    for jd in range(nch - d, nch):
        qd = jd % q
        pltpu.make_async_copy(table.at[src_v.at[0]], rows.at[qd],
                              gsems[qd]).wait()
        pltpu.async_copy(rows.at[qd], acc.at[dst_v.at[jd]],
                         ssems[qd], add=True)
        if ones_v is not None:
            pltpu.async_copy(ones_v, accc.at[dst_v.at[jd]], csem, add=True)
    for b in range(q):
        pltpu.make_async_copy(rows.at[b], acc.at[dst_v.at[0]],
                              ssems[b]).wait()
    if ones_v is not None:
        def drain(j, carry):
            pltpu.make_async_copy(ones_v, accc.at[dst_v.at[0]],
                                  csem).wait()
            return carry
        lax.fori_loop(0, nch, drain, 0)

@functools.lru_cache(maxsize=None)
def _sc_segsum(nch, a_pad, with_counts, a16_pad=0, stage_rows=0, q=_Q, d=_D):
    zrows = a_pad // 16
    out_type = [jax.ShapeDtypeStruct((2, a_pad, 64), f32)]
    scratch = [
        pltpu.VMEM((nch, 128), i32),
        pltpu.VMEM((nch, 128), i32),
        pltpu.VMEM((q, 128, 64), f32),
    ] + [pltpu.SemaphoreType.DMA] * (2 * q) + [
        pltpu.VMEM_SHARED((a_pad, 64), f32),
    ]
    if stage_rows:
        scratch.append(pltpu.VMEM_SHARED((stage_rows, 64), f32))
    if with_counts:
        z16 = a16_pad // 16
        out_type.append(jax.ShapeDtypeStruct((2, a16_pad, 16), f32))
        scratch += [
            pltpu.VMEM((128, 16), f32),
            pltpu.SemaphoreType.DMA,
            pltpu.VMEM_SHARED((a16_pad, 16), f32),
        ]

    def body(*refs):
        if with_counts:
            (table, src2d, dst2d, zeros64, zeros16, ones16,
             out, outc, src_v, dst_v, rows, *rest) = refs
            gsems, ssems = rest[:q], rest[q:2 * q]
            if stage_rows:
                acc, table_sh, ones_v, csem, accc = rest[2 * q:]
            else:
                acc, ones_v, csem, accc = rest[2 * q:]
                table_sh = None
        else:
            (table, src2d, dst2d, zeros64,
             out, src_v, dst_v, rows, *rest) = refs
            gsems, ssems = rest[:q], rest[q:2 * q]
            acc = rest[2 * q]
            table_sh = rest[2 * q + 1] if stage_rows else None
            ones_v = csem = accc = None
        cid = lax.axis_index("c")
        sid = lax.axis_index("s")
        wid = cid * 16 + sid
        if stage_rows:
            srows = stage_rows // 16
            pltpu.sync_copy(table.at[pl.ds(sid * srows, srows)],
                            table_sh.at[pl.ds(sid * srows, srows)])
            table = table_sh
        pltpu.sync_copy(zeros64.at[pl.ds(0, zrows)],
                        acc.at[pl.ds(sid * zrows, zrows)])
        if with_counts:
            z16 = a16_pad // 16
            pltpu.sync_copy(zeros16.at[pl.ds(0, z16)],
                            accc.at[pl.ds(sid * z16, z16)])
            pltpu.sync_copy(ones16, ones_v)
        pltpu.sync_copy(src2d.at[pl.ds(wid * nch, nch)], src_v)
        pltpu.sync_copy(dst2d.at[pl.ds(wid * nch, nch)], dst_v)
        plsc.subcore_barrier()
        _edge_loop(table, src_v, dst_v, rows, acc, gsems, ssems, nch,
                   ones_v=ones_v, accc=accc, csem=csem, q=q, d=d)
        plsc.subcore_barrier()
        pltpu.sync_copy(acc.at[pl.ds(sid * zrows, zrows)],
                        out.at[cid].at[pl.ds(sid * zrows, zrows)])
        if with_counts:
            z16 = a16_pad // 16
            pltpu.sync_copy(accc.at[pl.ds(sid * z16, z16)],
                            outc.at[cid].at[pl.ds(sid * z16, z16)])

    k = pl.kernel(body, out_type=out_type, mesh=_MESH, scratch_types=scratch,
                  compiler_params=_SC_PARAMS)
    if with_counts:
        return k
    return lambda *a: k(*a)[0]


@functools.lru_cache(maxsize=None)
def _sc_seg_percfg(nch):
    a_pad = 16 * NC + 16
    zrows = a_pad // 16

    def body(table, src2d, dst2d, zeros, out, src_v, dst_v, rows, *rest):
        gsems, ssems = rest[:_Q], rest[_Q:2 * _Q]
        acc = rest[2 * _Q]
        cid = lax.axis_index("c")
        sid = lax.axis_index("s")
        wid = cid * 16 + sid
        pltpu.sync_copy(zeros.at[pl.ds(0, zrows)],
                        acc.at[pl.ds(sid * zrows, zrows)])
        pltpu.sync_copy(src2d.at[pl.ds(wid * nch, nch)], src_v)
        pltpu.sync_copy(dst2d.at[pl.ds(sid * nch, nch)], dst_v)
        plsc.subcore_barrier()
        _edge_loop(table, src_v, dst_v, rows, acc, gsems, ssems, nch)
        plsc.subcore_barrier()
        pltpu.sync_copy(acc.at[pl.ds(sid * NC, NC)],
                        out.at[pl.ds(wid * NC, NC)])

    return pl.kernel(
        body,
        out_type=jax.ShapeDtypeStruct((C * NC, 64), f32),
        mesh=_MESH,
        scratch_types=[
            pltpu.VMEM((nch, 128), i32),
            pltpu.VMEM((nch, 128), i32),
            pltpu.VMEM((_Q, 128, 64), f32),
        ] + [pltpu.SemaphoreType.DMA] * (2 * _Q) + [
            pltpu.VMEM_SHARED((a_pad, 64), f32),
        ],
        compiler_params=_SC_PARAMS,
    )


@functools.lru_cache(maxsize=None)
def _sc_seg_final():
    nch = E_PAD // 4096
    zrows = A_PAD // 16
    srows = N // 16

    def body(table, src2d, dst2d, zeros64, ids2d, cnt0h, cnt1h,
             pg, xg, c0g, c1g, src_v, dst_v, rows, idx_v, cbuf, *rest):
        gsems, ssems = rest[:_Q], rest[_Q:2 * _Q]
        acc, table_sh = rest[2 * _Q], rest[2 * _Q + 1]
        cid = lax.axis_index("c")
        sid = lax.axis_index("s")
        wid = cid * 16 + sid
        pltpu.sync_copy(table.at[pl.ds(sid * srows, srows)],
                        table_sh.at[pl.ds(sid * srows, srows)])
        pltpu.sync_copy(zeros64.at[pl.ds(0, zrows)],
                        acc.at[pl.ds(sid * zrows, zrows)])
        pltpu.sync_copy(src2d.at[pl.ds(wid * nch, nch)], src_v)
        pltpu.sync_copy(dst2d.at[pl.ds(wid * nch, nch)], dst_v)
        plsc.subcore_barrier()
        _edge_loop(table_sh, src_v, dst_v, rows, acc, gsems, ssems, nch)
        plsc.subcore_barrier()

        nids = IDS_PAD // 128

        @pl.when(sid < nids)
        def _():
            pltpu.sync_copy(ids2d.at[pl.ds(sid, 1)], idx_v)
            pltpu.async_copy(acc.at[idx_v.at[0]], rows.at[0],
                             gsems[0]).wait()
            pltpu.sync_copy(rows.at[0],
                            pg.at[cid].at[pl.ds(sid * 128, 128)])

            @pl.when(cid == 0)
            def _():
                pltpu.async_copy(table_sh.at[idx_v.at[0]], rows.at[1],
                                 gsems[1]).wait()
                pltpu.sync_copy(rows.at[1], xg.at[pl.ds(sid * 128, 128)])

        @pl.when((sid >= nids) & (sid < 2 * nids))
        def _():
            s8 = sid - nids
            pltpu.sync_copy(ids2d.at[pl.ds(s8, 1)], idx_v)

            @pl.when(cid == 0)
            def _():
                pltpu.async_copy(cnt0h.at[idx_v.at[0]], cbuf,
                                 gsems[2]).wait()
                pltpu.sync_copy(cbuf, c0g.at[pl.ds(s8 * 128, 128)])

            @pl.when(cid == 1)
            def _():
                pltpu.async_copy(cnt1h.at[idx_v.at[0]], cbuf,
                                 gsems[2]).wait()
                pltpu.sync_copy(cbuf, c1g.at[pl.ds(s8 * 128, 128)])

    return pl.kernel(
        body,
        out_type=[
            jax.ShapeDtypeStruct((2, IDS_PAD, 64), f32),
            jax.ShapeDtypeStruct((IDS_PAD, 64), f32),
            jax.ShapeDtypeStruct((IDS_PAD, 16), f32),
            jax.ShapeDtypeStruct((IDS_PAD, 16), f32),
        ],
        mesh=_MESH,
        scratch_types=[
            pltpu.VMEM((nch, 128), i32),
            pltpu.VMEM((nch, 128), i32),
            pltpu.VMEM((_Q, 128, 64), f32),
            pltpu.VMEM((1, 128), i32),
            pltpu.VMEM((128, 16), f32),
        ] + [pltpu.SemaphoreType.DMA] * (2 * _Q) + [
            pltpu.VMEM_SHARED((A_PAD, 64), f32),
            pltpu.VMEM_SHARED((N, 64), f32),
        ],
        compiler_params=_SC_PARAMS,
    )


@functools.lru_cache(maxsize=None)
def _sc_gather(nrows, d):
    rpt = max(nrows // 32, 1)

    def body(table, idx2d, out, idx_v, rows_v, sem):
        cid = lax.axis_index("c")
        sid = lax.axis_index("s")
        wid = cid * 16 + sid
        for r in range(rpt):
            j = wid * rpt + r

            @pl.when(j < nrows)
            def _():
                pltpu.sync_copy(idx2d.at[pl.ds(j, 1)], idx_v)
                pltpu.async_copy(table.at[idx_v.at[0]], rows_v, sem).wait()
                pltpu.sync_copy(rows_v, out.at[pl.ds(j * 128, 128)])

    return pl.kernel(
        body,
        out_type=jax.ShapeDtypeStruct((nrows * 128, d), f32),
        mesh=_MESH,
        compiler_params=_SC_PARAMS,
        scratch_types=[
            pltpu.VMEM((1, 128), i32),
            pltpu.VMEM((128, d), f32),
            pltpu.SemaphoreType.DMA,
        ],
    )


def _dot(a, b):
    return jnp.dot(a, b, preferred_element_type=f32)


def _tc_proj2(nf, eg, wla, wlb, wra, wrb):
    def body(nf_r, eg_r, wla_r, wlb_r, wra_r, wrb_r, l_r, r_r):
        a = nf_r[...]
        e = eg_r[...]
        l_r[...] = _dot(a, wla_r[...]) + _dot(e, wlb_r[...])
        r_r[...] = _dot(a, wra_r[...]) + _dot(e, wrb_r[...])

    n = nf.shape[0]
    return pl.pallas_call(
        body,
        out_shape=[jax.ShapeDtypeStruct((n, 64), f32)] * 2,
    )(nf, eg, wla, wlb, wra, wrb)


def _tc_proj(x, wl, wr):
    def body(x_r, wl_r, wr_r, l_r, r_r):
        v = x_r[...]
        l_r[...] = _dot(v, wl_r[...])
        r_r[...] = _dot(v, wr_r[...])

    n = x.shape[0]
    return pl.pallas_call(
        body,
        out_shape=[jax.ShapeDtypeStruct((n, 64), f32)] * 2,
    )(x, wl, wr)


def _tc_combine(s0, s1, c0, c1, r, bl, wl=None, wr=None):
    project = wl is not None

    def body(*refs):
        if project:
            s0_r, s1_r, c0_r, c1_r, r_r, bl_r, wl_r, wr_r, *outs = refs
        else:
            s0_r, s1_r, c0_r, c1_r, r_r, bl_r, *outs = refs
        cnt = jnp.maximum(c0_r[...][:, :1] + c1_r[...][:, :1], 1.0)
        x = (s0_r[...] + s1_r[...]) / cnt + bl_r[...] + r_r[...]
        x = _leak(x)
        if project:
            outs[0][...] = _dot(x, wl_r[...])
            outs[1][...] = _dot(x, wr_r[...])
        else:
            outs[0][...] = x

    n = s0.shape[0]
    if project:
        return pl.pallas_call(
            body, out_shape=[jax.ShapeDtypeStruct((n, 64), f32)] * 2,
        )(s0, s1, c0, c1, r, bl, wl, wr)
    return pl.pallas_call(
        body, out_shape=jax.ShapeDtypeStruct((n, 64), f32),
    )(s0, s1, c0, c1, r, bl)


def _tc_xcat(x3, p0, p1, c0, c1):
    def body(x_r, p0_r, p1_r, c0_r, c1_r, o_r):
        cnt = jnp.maximum(c0_r[...][:, :1] + c1_r[...][:, :1], 1.0)
        o_r[:, :64] = x_r[...]
        o_r[:, 64:] = (p0_r[...] + p1_r[...]) / cnt

    return pl.pallas_call(
        body, out_shape=jax.ShapeDtypeStruct((N, 128), f32),
    )(x3, p0, p1, c0, c1)


def _tc_cfgstart(p0, p1, c0, c1, wl, wr):
    def body(p0_r, p1_r, c0_r, c1_r, wl_r, wr_r, l_r, r_r):
        cnt = jnp.maximum(c0_r[...][:, :1] + c1_r[...][:, :1], 1.0)
        nb = (p0_r[...] + p1_r[...]) / cnt
        l_r[...] = _dot(nb, wl_r[...])
        r_r[...] = _dot(nb, wr_r[...])

    n = p0.shape[0]
    return pl.pallas_call(
        body, out_shape=[jax.ShapeDtypeStruct((n, 64), f32)] * 2,
    )(p0, p1, c0, c1, wl, wr)


def _tc_cfgprep(ncfeat, t0, t1, cc0, cc1, t_r, cbl3, xc, prjw, prjb,
                wl_a, wl_b, wl_c, wr_a, wr_b, wr_c):
    def body(nc_r, t0_r, t1_r, cc0_r, cc1_r, tr_r, cbl_r, xc_r, pw_r, pb_r,
             wla_r, wlb_r, wlc_r, wra_r, wrb_r, wrc_r, yl_r, yr_r):
        cnt = jnp.maximum(cc0_r[...][:, :1] + cc1_r[...][:, :1], 1.0)
        nbv = _leak((t0_r[...] + t1_r[...]) / cnt + cbl_r[...] + tr_r[...])
        xcv = xc_r[...]
        ncf = _leak(_dot(nc_r[0], pw_r[...]) + pb_r[...])
        n2 = (jnp.sum(nbv * nbv, axis=1, keepdims=True)
              + jnp.sum(xcv * xcv, axis=1, keepdims=True)
              + jnp.sum(ncf * ncf, axis=1, keepdims=True))
        inv = 1.0 / jnp.maximum(jnp.sqrt(n2), 1e-12)
        al = _dot(nbv, wla_r[...]) + _dot(xcv, wlb_r[...])
        ar = _dot(nbv, wra_r[...]) + _dot(xcv, wrb_r[...])
        yl_r[0] = (al + _dot(ncf, wlc_r[...])) * inv
        yr_r[0] = (ar + _dot(ncf, wrc_r[...])) * inv

    full = lambda shp: pl.BlockSpec(shp, lambda c: tuple(0 for _ in shp))
    cfg_blk = pl.BlockSpec((1, NC, 64), lambda c: (c, 0, 0))
    return pl.pallas_call(
        body,
        grid=(C,),
        in_specs=[
            pl.BlockSpec((1, NC, 18), lambda c: (c, 0, 0)),
            full((NC, 64)), full((NC, 64)), full((NC, 16)), full((NC, 16)),
            full((NC, 64)), full((1, 64)), full((NC, 64)),
            full((18, 64)), full((1, 64)),
            full((64, 64)), full((64, 64)), full((64, 64)),
            full((64, 64)), full((64, 64)), full((64, 64)),
        ],
        out_specs=[cfg_blk] * 2,
        out_shape=[jax.ShapeDtypeStruct((C, NC, 64), f32)] * 2,
    )(ncfeat, t0, t1, cc0, cc1, t_r, cbl3, xc, prjw, prjb,
      wl_a, wl_b, wl_c, wr_a, wr_b, wr_c)


def _tc_cfg_combine(s, c0, c1, yr, bl, wl, wr):
    def body(s_r, c0_r, c1_r, yr_r, bl_r, wl_r, wr_r, l_r, r_r):
        cnt = jnp.maximum(c0_r[...][:, :1] + c1_r[...][:, :1], 1.0)
        x = s_r[0] / cnt + bl_r[...] + yr_r[0]
        x = _leak(x)
        l_r[0] = _dot(x, wl_r[...])
        r_r[0] = _dot(x, wr_r[...])

    full = lambda shp: pl.BlockSpec(shp, lambda c: tuple(0 for _ in shp))
    cfg_blk = pl.BlockSpec((1, NC, 64), lambda c: (c, 0, 0))
    return pl.pallas_call(
        body,
        grid=(C,),
        in_specs=[cfg_blk, full((NC, 16)), full((NC, 16)), cfg_blk,
                  full((1, 64)), full((64, 64)), full((64, 64))],
        out_specs=[cfg_blk] * 2,
        out_shape=[jax.ShapeDtypeStruct((C, NC, 64), f32)] * 2,
    )(s, c0, c1, yr, bl, wl, wr)


def _tc_final(s, c0, c1, yr, bl, d1, d2, d3):
    def pool_body(s_r, c0_r, c1_r, yr_r, bl_r, o_r):
        cnt = jnp.maximum(c0_r[...][:, :1] + c1_r[...][:, :1], 1.0)
        h = s_r[0] / cnt + bl_r[...] + yr_r[0]
        h = _leak(h)
        o_r[0] = jnp.mean(h, axis=0, keepdims=True)

    full = lambda shp: pl.BlockSpec(shp, lambda c: tuple(0 for _ in shp))
    cfg_blk = pl.BlockSpec((1, NC, 64), lambda c: (c, 0, 0))
    pooled = pl.pallas_call(
        pool_body,
        grid=(C,),
        in_specs=[cfg_blk, full((NC, 16)), full((NC, 16)), cfg_blk,
                  full((1, 64))],
        out_specs=pl.BlockSpec((1, 1, 64), lambda c: (c, 0, 0)),
        out_shape=jax.ShapeDtypeStruct((C, 1, 64), f32),
    )(s, c0, c1, yr, bl).reshape(C, 64)

    def head_body(p_r, d1_r, d2_r, d3_r, o_r):
        y = _leak(_dot(p_r[...], d1_r[...]))
        y = _leak(_dot(y, d2_r[...]))
        o_r[...] = _dot(y, d3_r[...])

    return pl.pallas_call(
        head_body, out_shape=jax.ShapeDtypeStruct((C, 1), f32),
    )(pooled, d1, d2, d3)


def kernel(node_feat, node_opcode, edge_index, node_config_feat,
           node_config_ids, config_edge_index, params):
    p = params
    src = edge_index[0].astype(i32)
    dst = edge_index[1].astype(i32)
    csrc = config_edge_index[0].astype(i32)
    cdst = config_edge_index[1].astype(i32)
    opc = node_opcode.astype(i32)
    ids = node_config_ids.astype(i32)

    srcp = jnp.concatenate([src, jnp.zeros((E_PAD - E,), i32)]).reshape(-1, 128)
    dstp = jnp.concatenate([dst, jnp.full((E_PAD - E,), N, i32)]).reshape(-1, 128)
    opcp = jnp.concatenate([opc, jnp.zeros((OP_PAD - N,), i32)]).reshape(-1, 128)
    idsp = jnp.concatenate([ids, jnp.zeros((IDS_PAD - NC,), i32)]).reshape(-1, 128)
    csrcp = jnp.concatenate([csrc, jnp.zeros((EC_PAD - EC,), i32)]).reshape(-1, 128)
    cdstp = jnp.concatenate([cdst, jnp.full((EC_PAD - EC,), NC, i32)]).reshape(-1, 128)
    coff = (jnp.arange(C, dtype=i32) * NC)[:, None]
    gsrc = (csrcp.reshape(-1)[None, :] + coff).reshape(-1, 128)
    pad_mask = jnp.arange(EC_PAD) >= EC
    cdst0 = jnp.concatenate([cdst, jnp.zeros((EC_PAD - EC,), i32)])
    soff = (jnp.arange(16, dtype=i32) * NC)[:, None]
    dstcfg = jnp.where(pad_mask[None, :], 16 * NC,
                       cdst0[None, :] + soff).reshape(-1, 128)

    zeros64 = jnp.zeros((AG_PAD // 16, 64), f32)
    zeros16 = jnp.zeros((A_PAD // 16, 16), f32)
    ones16 = jnp.ones((128, 16), f32)

    ng, cg, gg = p["node_gnn"], p["cfg_nbr_gnn"], p["cfg_gnn"]
    wl1, wr1 = ng[0]["Wl"], ng[0]["Wr"]
    bl = [lyr["bl"].reshape(1, 64) for lyr in ng]
    cbl = [lyr["bl"].reshape(1, 64) for lyr in cg]
    gbl = [lyr["bl"].reshape(1, 64) for lyr in gg]
    gwl1, gwr1 = gg[0]["Wl"], gg[0]["Wr"]

    embg = _sc_gather(OP_PAD // 128, 32)(p["embedding"], opcp)[:N]
    l1, r1 = _tc_proj2(node_feat, embg, wl1[:140], wl1[140:], wr1[:140], wr1[140:])
    s1, cnt = _sc_segsum(E_PAD // 4096, A_PAD, True, A_PAD, N, 2, 1)(
        l1, srcp, dstp, zeros64, zeros16, ones16)
    c0, c1 = cnt[0, :N], cnt[1, :N]
    l2, r2 = _tc_combine(s1[0, :N], s1[1, :N], c0, c1, r1, bl[0],
                         ng[1]["Wl"], ng[1]["Wr"])
    s2 = _sc_segsum(E_PAD // 4096, A_PAD, False, 0, N)(l2, srcp, dstp, zeros64)
    l3, r3 = _tc_combine(s2[0, :N], s2[1, :N], c0, c1, r2, bl[1],
                         ng[2]["Wl"], ng[2]["Wr"])
    s3 = _sc_segsum(E_PAD // 4096, A_PAD, False, 0, N)(l3, srcp, dstp, zeros64)
    x3 = _tc_combine(s3[0, :N], s3[1, :N], c0, c1, r3, bl[2])
    pg, xg, c0gf, c1gf = _sc_seg_final()(
        x3, srcp, dstp, zeros64, idsp, cnt[0], cnt[1])
    xc = xg[:NC]

    t_l, t_r = _tc_cfgstart(pg[0, :NC], pg[1, :NC], c0gf[:NC], c1gf[:NC],
                            cg[0]["Wl"], cg[0]["Wr"])
    t1, ccnt = _sc_segsum(EC_PAD // 4096, AC_PAD, True, AC_PAD)(
        t_l, csrcp, cdstp, zeros64, zeros16, ones16)
    cc0, cc1 = ccnt[0, :NC], ccnt[1, :NC]
    t_l, t_r = _tc_combine(t1[0, :NC], t1[1, :NC], cc0, cc1, t_r, cbl[0],
                           cg[1]["Wl"], cg[1]["Wr"])
    t2 = _sc_segsum(EC_PAD // 4096, AC_PAD, False)(t_l, csrcp, cdstp, zeros64)
    t_l, t_r = _tc_combine(t2[0, :NC], t2[1, :NC], cc0, cc1, t_r, cbl[1],
                           cg[2]["Wl"], cg[2]["Wr"])
    t3 = _sc_segsum(EC_PAD // 4096, AC_PAD, False)(t_l, csrcp, cdstp, zeros64)

    yl, yr = _tc_cfgprep(node_config_feat, t3[0, :NC], t3[1, :NC], cc0, cc1,
                         t_r, cbl[2], xc,
                         p["prj_W"], p["prj_b"].reshape(1, 64),
                         gwl1[:64], gwl1[64:128], gwl1[128:],
                         gwr1[:64], gwr1[64:128], gwr1[128:])
    g1 = _sc_seg_percfg(EG_PAD // 4096)(
        yl.reshape(C * NC, 64), gsrc, dstcfg, zeros64).reshape(C, NC, 64)
    yl, yr = _tc_cfg_combine(g1, cc0, cc1, yr, gbl[0],
                             gg[1]["Wl"], gg[1]["Wr"])
    g2 = _sc_seg_percfg(EG_PAD // 4096)(
        yl.reshape(C * NC, 64), gsrc, dstcfg, zeros64).reshape(C, NC, 64)
    yl, yr = _tc_cfg_combine(g2, cc0, cc1, yr, gbl[1],
                             gg[2]["Wl"], gg[2]["Wr"])
    g3 = _sc_seg_percfg(EG_PAD // 4096)(
        yl.reshape(C * NC, 64), gsrc, dstcfg, zeros64).reshape(C, NC, 64)
    y = _tc_final(g3, cc0, cc1, yr, gbl[2],
                  p["d1"], p["d2"], p["d3"])
    return y.reshape(-1)
